# Initial kernel scaffold; baseline (speedup 1.0000x reference)
import functools

import jax
import jax.numpy as jnp
from jax import lax
from jax.experimental import pallas as pl
from jax.experimental.pallas import tpu as pltpu
from jax.experimental.pallas import tpu_sc as plsc

N = 50000
E = 800000
DIN = 26
H = 128
CH = 32
NCH = H // CH
NSUB = 16
NCORE = 2
W = 128
EP = 819200
PAD = EP - E
TRASH = 64
SLAB_ROWS = N + TRASH
STRIPE = N // NSUB
ZR = 625


def _sc_agg(table, srcp, dstp, hid):
    mesh = plsc.VectorSubcoreMesh(core_axis_name="c", subcore_axis_name="s")
    if hid:
        nout = NCH
        ept = EP // NSUB
    else:
        nout = 2
        ept = EP // (NSUB * NCORE)
    nsteps = ept // W

    @functools.partial(
        pl.kernel,
        mesh=mesh,
        out_type=jax.ShapeDtypeStruct((nout, N, CH), jnp.float32),
        scratch_types=[
            pltpu.VMEM_SHARED((SLAB_ROWS, CH), jnp.float32),
            pltpu.VMEM((W,), jnp.int32),
            pltpu.VMEM((1, W), jnp.int32),
            pltpu.VMEM((ZR, CH), jnp.float32),
            pltpu.VMEM((W, CH), jnp.float32),
            pltpu.SemaphoreType.DMA,
        ],
    )
    def k(tab_hbm, src_hbm, dst_hbm, out_hbm, slab, sidx, didx, zbuf, rows, sem):
        cid = lax.axis_index("c")
        sid = lax.axis_index("s")

        @pl.loop(0, ZR)
        def _(r):
            zbuf[r, pl.ds(0, 16)] = jnp.zeros((16,), jnp.float32)
            zbuf[r, pl.ds(16, 16)] = jnp.zeros((16,), jnp.float32)

        for p in range(2 if hid else 1):
            if hid:
                chunk = cid * 2 + p
                oslot = chunk
                ebase = sid * ept
            else:
                chunk = None
                oslot = cid
                ebase = (sid * NCORE + cid) * ept

            plsc.subcore_barrier()
            for z in range(STRIPE // ZR):
                pltpu.sync_copy(zbuf, slab.at[pl.ds(sid * STRIPE + z * ZR, ZR)])
            plsc.subcore_barrier()

            @pl.loop(0, nsteps)
            def _(stp):
                off = ebase + stp * W
                pltpu.sync_copy(src_hbm.at[pl.ds(off, W)], sidx)
                pltpu.sync_copy(dst_hbm.at[pl.ds(off, W)], didx.at[0])
                if hid:
                    @pl.loop(0, W, step=16)
                    def _(i):
                        sidx[pl.ds(i, 16)] = sidx[pl.ds(i, 16)] * NCH + chunk
                pltpu.async_copy(tab_hbm.at[sidx], rows, sem).wait()
                pltpu.sync_copy(rows, slab.at[didx.at[0]], add=True)

            plsc.subcore_barrier()
            pltpu.sync_copy(
                slab.at[pl.ds(sid * STRIPE, STRIPE)],
                out_hbm.at[oslot, pl.ds(sid * STRIPE, STRIPE)],
            )

    return k(table, srcp, dstp)


_DOT = dict(
    dimension_numbers=(((1,), (0,)), ((), ())),
    preferred_element_type=jnp.float32,
    precision=jax.lax.Precision.HIGHEST,
)

_R = 2000


def _l1_body(xp, p0, p1, wl, wr, bb, h_out, cnt_out):
    p = p0[...] + p1[...]
    cnt = p[:, DIN:DIN + 1]
    mean = p * (1.0 / jnp.maximum(cnt, 1.0))
    acc = lax.dot_general(mean, wl[...], **_DOT)
    acc = acc + lax.dot_general(xp[...], wr[...], **_DOT)
    h_out[...] = jnp.maximum(acc + bb[...], 0.0)
    cnt_out[...] = cnt


def _l1_dense(xpad, p0, p1, wlT, wrT, b):
    return pl.pallas_call(
        _l1_body,
        grid=(N // _R,),
        in_specs=[
            pl.BlockSpec((_R, CH), lambda i: (i, 0)),
            pl.BlockSpec((_R, CH), lambda i: (i, 0)),
            pl.BlockSpec((_R, CH), lambda i: (i, 0)),
            pl.BlockSpec((CH, H), lambda i: (0, 0)),
            pl.BlockSpec((CH, H), lambda i: (0, 0)),
            pl.BlockSpec((1, H), lambda i: (0, 0)),
        ],
        out_specs=[
            pl.BlockSpec((_R, H), lambda i: (i, 0)),
            pl.BlockSpec((_R, 1), lambda i: (i, 0)),
        ],
        out_shape=[
            jax.ShapeDtypeStruct((N, H), jnp.float32),
            jax.ShapeDtypeStruct((N, 1), jnp.float32),
        ],
    )(xpad, p0, p1, wlT, wrT, b)


def _hid_body(agg, cnt, h, wl, wr, bb, h_out):
    mean = agg[...] * (1.0 / jnp.maximum(cnt[...], 1.0))
    acc = lax.dot_general(mean, wl[...], **_DOT)
    acc = acc + lax.dot_general(h[...], wr[...], **_DOT)
    h_out[...] = jnp.maximum(acc + bb[...], 0.0)


def _hid_dense(agg, cnt, h, wlT, wrT, b):
    return pl.pallas_call(
        _hid_body,
        grid=(N // _R,),
        in_specs=[
            pl.BlockSpec((_R, H), lambda i: (i, 0)),
            pl.BlockSpec((_R, 1), lambda i: (i, 0)),
            pl.BlockSpec((_R, H), lambda i: (i, 0)),
            pl.BlockSpec((H, H), lambda i: (0, 0)),
            pl.BlockSpec((H, H), lambda i: (0, 0)),
            pl.BlockSpec((1, H), lambda i: (0, 0)),
        ],
        out_specs=pl.BlockSpec((_R, H), lambda i: (i, 0)),
        out_shape=jax.ShapeDtypeStruct((N, H), jnp.float32),
    )(agg, cnt, h, wlT, wrT, b)


def kernel(x, edge_index, W1l, b1, W1r, W2l, b2, W2r, W3l, b3, W3r):
    src = edge_index[0]
    dst = edge_index[1]
    ar = jnp.arange(PAD, dtype=jnp.int32)
    srcp = jnp.concatenate([src, (ar * 31) % N])
    dstp = jnp.concatenate([dst, N + (ar & (TRASH - 1))])

    xpad = jnp.concatenate(
        [x, jnp.ones((N, 1), jnp.float32), jnp.zeros((N, CH - DIN - 1), jnp.float32)],
        axis=1,
    )
    w1lT = jnp.pad(W1l, ((0, 0), (0, CH - DIN))).T
    w1rT = jnp.pad(W1r, ((0, 0), (0, CH - DIN))).T

    p = _sc_agg(xpad, srcp, dstp, hid=False)
    h, cnt = _l1_dense(xpad, p[0], p[1], w1lT, w1rT, b1.reshape(1, H))

    for Wl, b, Wr in ((W2l, b2, W2r), (W3l, b3, W3r)):
        agg4 = _sc_agg(h.reshape(NCH * N, CH), srcp, dstp, hid=True)
        agg = jnp.transpose(agg4, (1, 0, 2)).reshape(N, H)
        h = _hid_dense(agg, cnt, h, Wl.T, Wr.T, b.reshape(1, H))
    return h


# speedup vs baseline: 3.0949x; 3.0949x over previous
import functools

import jax
import jax.numpy as jnp
from jax import lax
from jax.experimental import pallas as pl
from jax.experimental.pallas import tpu as pltpu
from jax.experimental.pallas import tpu_sc as plsc

N = 50000
E = 800000
DIN = 26
H = 128
CH = 32
NCH = H // CH
NSUB = 16
NCORE = 2
W = 128
EP = 819200
PAD = EP - E
TRASH = 64
NPAD = 50176
SLAB_ROWS = NPAD + TRASH
STRIPE = NPAD // NSUB
ZR = 784


def _sc_agg(table, srcp, dstp, hid):
    mesh = plsc.VectorSubcoreMesh(core_axis_name="c", subcore_axis_name="s")
    if hid:
        nout = NCH
        ept = EP // NSUB
    else:
        nout = 2
        ept = EP // (NSUB * NCORE)
    nsteps = ept // W

    @functools.partial(
        pl.kernel,
        mesh=mesh,
        compiler_params=pltpu.CompilerParams(use_tc_tiling_on_sc=False),
        out_type=jax.ShapeDtypeStruct((nout, NPAD, CH), jnp.float32),
        scratch_types=[
            pltpu.VMEM_SHARED((SLAB_ROWS, CH), jnp.float32),
            pltpu.VMEM((W,), jnp.int32),
            pltpu.VMEM((1, W), jnp.int32),
            pltpu.VMEM((ZR, CH), jnp.float32),
            pltpu.VMEM((W, CH), jnp.float32),
            pltpu.SemaphoreType.DMA,
        ],
    )
    def k(tab_hbm, src_hbm, dst_hbm, out_hbm, slab, sidx, didx, zbuf, rows, sem):
        cid = lax.axis_index("c")
        sid = lax.axis_index("s")

        @pl.loop(0, ZR)
        def _(r):
            zbuf[r, pl.ds(0, 16)] = jnp.zeros((16,), jnp.float32)
            zbuf[r, pl.ds(16, 16)] = jnp.zeros((16,), jnp.float32)

        for p in range(2 if hid else 1):
            if hid:
                chunk = cid * 2 + p
                oslot = chunk
                ebase = sid * ept
            else:
                chunk = None
                oslot = cid
                ebase = (sid * NCORE + cid) * ept

            plsc.subcore_barrier()
            for z in range(STRIPE // ZR):
                pltpu.sync_copy(zbuf, slab.at[pl.ds(sid * STRIPE + z * ZR, ZR)])
            plsc.subcore_barrier()

            @pl.loop(0, nsteps)
            def _(stp):
                off = ebase + stp * W
                pltpu.sync_copy(src_hbm.at[pl.ds(off, W)], sidx)
                pltpu.sync_copy(dst_hbm.at[pl.ds(off, W)], didx.at[0])
                if hid:
                    @pl.loop(0, W, step=16)
                    def _(i):
                        sidx[pl.ds(i, 16)] = sidx[pl.ds(i, 16)] * NCH + chunk
                pltpu.async_copy(tab_hbm.at[sidx], rows, sem).wait()
                pltpu.sync_copy(rows, slab.at[didx.at[0]], add=True)

            plsc.subcore_barrier()
            pltpu.sync_copy(
                slab.at[pl.ds(sid * STRIPE, STRIPE)],
                out_hbm.at[oslot, pl.ds(sid * STRIPE, STRIPE)],
            )

    return k(table, srcp, dstp)


_DOT = dict(
    dimension_numbers=(((1,), (0,)), ((), ())),
    preferred_element_type=jnp.float32,
    precision=jax.lax.Precision.HIGHEST,
)

_R = 2000


def _l1_body(xp, p0, p1, wl, wr, bb, h_out, cnt_out):
    p = p0[...] + p1[...]
    cnt = p[:, DIN:DIN + 1]
    mean = p * (1.0 / jnp.maximum(cnt, 1.0))
    acc = lax.dot_general(mean, wl[...], **_DOT)
    acc = acc + lax.dot_general(xp[...], wr[...], **_DOT)
    h_out[...] = jnp.maximum(acc + bb[...], 0.0)
    cnt_out[...] = cnt


def _l1_dense(xpad, p0, p1, wlT, wrT, b):
    return pl.pallas_call(
        _l1_body,
        grid=(N // _R,),
        in_specs=[
            pl.BlockSpec((_R, CH), lambda i: (i, 0)),
            pl.BlockSpec((_R, CH), lambda i: (i, 0)),
            pl.BlockSpec((_R, CH), lambda i: (i, 0)),
            pl.BlockSpec((CH, H), lambda i: (0, 0)),
            pl.BlockSpec((CH, H), lambda i: (0, 0)),
            pl.BlockSpec((1, H), lambda i: (0, 0)),
        ],
        out_specs=[
            pl.BlockSpec((_R, H), lambda i: (i, 0)),
            pl.BlockSpec((_R, 1), lambda i: (i, 0)),
        ],
        out_shape=[
            jax.ShapeDtypeStruct((N, H), jnp.float32),
            jax.ShapeDtypeStruct((N, 1), jnp.float32),
        ],
    )(xpad, p0, p1, wlT, wrT, b)


def _hid_body(agg, cnt, h, wl, wr, bb, h_out):
    mean = agg[...] * (1.0 / jnp.maximum(cnt[...], 1.0))
    acc = lax.dot_general(mean, wl[...], **_DOT)
    acc = acc + lax.dot_general(h[...], wr[...], **_DOT)
    h_out[...] = jnp.maximum(acc + bb[...], 0.0)


def _hid_dense(agg, cnt, h, wlT, wrT, b):
    return pl.pallas_call(
        _hid_body,
        grid=(N // _R,),
        in_specs=[
            pl.BlockSpec((_R, H), lambda i: (i, 0)),
            pl.BlockSpec((_R, 1), lambda i: (i, 0)),
            pl.BlockSpec((_R, H), lambda i: (i, 0)),
            pl.BlockSpec((H, H), lambda i: (0, 0)),
            pl.BlockSpec((H, H), lambda i: (0, 0)),
            pl.BlockSpec((1, H), lambda i: (0, 0)),
        ],
        out_specs=pl.BlockSpec((_R, H), lambda i: (i, 0)),
        out_shape=jax.ShapeDtypeStruct((N, H), jnp.float32),
    )(agg, cnt, h, wlT, wrT, b)


def kernel(x, edge_index, W1l, b1, W1r, W2l, b2, W2r, W3l, b3, W3r):
    src = edge_index[0]
    dst = edge_index[1]
    ar = jnp.arange(PAD, dtype=jnp.int32)
    srcp = jnp.concatenate([src, (ar * 31) % N])
    dstp = jnp.concatenate([dst, NPAD + (ar & (TRASH - 1))])

    xpad = jnp.concatenate(
        [x, jnp.ones((N, 1), jnp.float32), jnp.zeros((N, CH - DIN - 1), jnp.float32)],
        axis=1,
    )
    w1lT = jnp.pad(W1l, ((0, 0), (0, CH - DIN))).T
    w1rT = jnp.pad(W1r, ((0, 0), (0, CH - DIN))).T

    p = _sc_agg(xpad, srcp, dstp, hid=False)[:, :N]
    h, cnt = _l1_dense(xpad, p[0], p[1], w1lT, w1rT, b1.reshape(1, H))

    for Wl, b, Wr in ((W2l, b2, W2r), (W3l, b3, W3r)):
        agg4 = _sc_agg(h.reshape(NCH * N, CH), srcp, dstp, hid=True)[:, :N]
        agg = jnp.transpose(agg4, (1, 0, 2)).reshape(N, H)
        h = _hid_dense(agg, cnt, h, Wl.T, Wr.T, b.reshape(1, H))
    return h


# speedup vs baseline: 7.9465x; 2.5676x over previous
import functools

import jax
import jax.numpy as jnp
from jax import lax
from jax.experimental import pallas as pl
from jax.experimental.pallas import tpu as pltpu
from jax.experimental.pallas import tpu_sc as plsc

N = 50000
E = 800000
DIN = 26
H = 128
CH = 32
NCH = H // CH
NSUB = 16
NCORE = 2
W = 128
GSTEPS = 5
GE = GSTEPS * W
EP = 819200
PAD = EP - E
TRASH = 64
NPAD = 50176
SLAB_ROWS = NPAD + TRASH
STRIPE = NPAD // NSUB
ZR = 196


def _sc_agg(table, srcp, dstp, hid):
    mesh = plsc.VectorSubcoreMesh(core_axis_name="c", subcore_axis_name="s")
    if hid:
        nout = NCH
        ept = EP // NSUB
    else:
        nout = 2
        ept = EP // (NSUB * NCORE)
    ngroups = ept // GE

    @functools.partial(
        pl.kernel,
        mesh=mesh,
        compiler_params=pltpu.CompilerParams(use_tc_tiling_on_sc=False),
        out_type=jax.ShapeDtypeStruct((nout, NPAD, CH), jnp.float32),
        scratch_types=[
            pltpu.VMEM_SHARED((SLAB_ROWS, CH), jnp.float32),
            pltpu.VMEM((2, GE), jnp.int32),
            pltpu.VMEM((2, GSTEPS, W), jnp.int32),
            pltpu.VMEM((ZR, CH), jnp.float32),
            pltpu.VMEM((GE, CH), jnp.float32),
        ] + [pltpu.SemaphoreType.DMA] * (GSTEPS + 4),
    )
    def k(tab_hbm, src_hbm, dst_hbm, out_hbm, slab, sidx, didx, zbuf, rows, *sems):
        gsem = sems[:GSTEPS]
        isem = sems[GSTEPS:]
        cid = lax.axis_index("c")
        sid = lax.axis_index("s")

        @pl.loop(0, ZR)
        def _(r):
            zbuf[r, pl.ds(0, 16)] = jnp.zeros((16,), jnp.float32)
            zbuf[r, pl.ds(16, 16)] = jnp.zeros((16,), jnp.float32)

        for p in range(2 if hid else 1):
            if hid:
                chunk = cid * 2 + p
                oslot = chunk
                ebase = sid * ept
            else:
                chunk = None
                oslot = cid
                ebase = (sid * NCORE + cid) * ept
            rbase = ebase // W

            plsc.subcore_barrier()
            for z in range(STRIPE // ZR):
                pltpu.sync_copy(zbuf, slab.at[pl.ds(sid * STRIPE + z * ZR, ZR)])
            plsc.subcore_barrier()

            def fire_idx(g, par):
                pltpu.async_copy(
                    src_hbm.at[pl.ds(ebase + g * GE, GE)], sidx.at[par],
                    isem[2 * par])
                pltpu.async_copy(
                    dst_hbm.at[pl.ds(rbase + g * GSTEPS, GSTEPS)], didx.at[par],
                    isem[2 * par + 1])

            def wait_idx(par):
                pltpu.make_async_copy(
                    src_hbm.at[pl.ds(ebase, GE)], sidx.at[par],
                    isem[2 * par]).wait()
                pltpu.make_async_copy(
                    dst_hbm.at[pl.ds(rbase, GSTEPS)], didx.at[par],
                    isem[2 * par + 1]).wait()

            def group_body(g, par, other):
                @pl.when(g + 1 < ngroups)
                def _():
                    fire_idx(g + 1, other)
                wait_idx(par)
                handles = []
                for b in range(GSTEPS):
                    if hid:
                        @pl.loop(0, W, step=16)
                        def _(i, _b=b, _p=par):
                            sl = (_p, pl.ds(_b * W + i, 16))
                            sidx[sl] = sidx[sl] * NCH + chunk
                    handles.append(pltpu.async_copy(
                        tab_hbm.at[sidx.at[par, pl.ds(b * W, W)]],
                        rows.at[pl.ds(b * W, W)], gsem[b]))
                for b in range(GSTEPS):
                    handles[b].wait()
                    pltpu.sync_copy(
                        rows.at[pl.ds(b * W, W)],
                        slab.at[didx.at[par, b]], add=True)

            fire_idx(0, 0)

            @pl.loop(0, ngroups // 2)
            def _(i):
                group_body(2 * i, 0, 1)
                group_body(2 * i + 1, 1, 0)

            plsc.subcore_barrier()
            pltpu.sync_copy(
                slab.at[pl.ds(sid * STRIPE, STRIPE)],
                out_hbm.at[oslot, pl.ds(sid * STRIPE, STRIPE)],
            )

    return k(table, srcp, dstp.reshape(EP // W, W))


_DOT = dict(
    dimension_numbers=(((1,), (0,)), ((), ())),
    preferred_element_type=jnp.float32,
    precision=jax.lax.Precision.HIGHEST,
)

_R = 2000


def _l1_body(xp, p0, p1, wl, wr, bb, h_out, cnt_out):
    p = p0[...] + p1[...]
    cnt = p[:, DIN:DIN + 1]
    mean = p * (1.0 / jnp.maximum(cnt, 1.0))
    acc = lax.dot_general(mean, wl[...], **_DOT)
    acc = acc + lax.dot_general(xp[...], wr[...], **_DOT)
    h_out[...] = jnp.maximum(acc + bb[...], 0.0)
    cnt_out[...] = cnt


def _l1_dense(xpad, p0, p1, wlT, wrT, b):
    return pl.pallas_call(
        _l1_body,
        grid=(N // _R,),
        in_specs=[
            pl.BlockSpec((_R, CH), lambda i: (i, 0)),
            pl.BlockSpec((_R, CH), lambda i: (i, 0)),
            pl.BlockSpec((_R, CH), lambda i: (i, 0)),
            pl.BlockSpec((CH, H), lambda i: (0, 0)),
            pl.BlockSpec((CH, H), lambda i: (0, 0)),
            pl.BlockSpec((1, H), lambda i: (0, 0)),
        ],
        out_specs=[
            pl.BlockSpec((_R, H), lambda i: (i, 0)),
            pl.BlockSpec((_R, 1), lambda i: (i, 0)),
        ],
        out_shape=[
            jax.ShapeDtypeStruct((N, H), jnp.float32),
            jax.ShapeDtypeStruct((N, 1), jnp.float32),
        ],
    )(xpad, p0, p1, wlT, wrT, b)


def _hid_body(agg, cnt, h, wl, wr, bb, h_out):
    mean = agg[...] * (1.0 / jnp.maximum(cnt[...], 1.0))
    acc = lax.dot_general(mean, wl[...], **_DOT)
    acc = acc + lax.dot_general(h[...], wr[...], **_DOT)
    h_out[...] = jnp.maximum(acc + bb[...], 0.0)


def _hid_dense(agg, cnt, h, wlT, wrT, b):
    return pl.pallas_call(
        _hid_body,
        grid=(N // _R,),
        in_specs=[
            pl.BlockSpec((_R, H), lambda i: (i, 0)),
            pl.BlockSpec((_R, 1), lambda i: (i, 0)),
            pl.BlockSpec((_R, H), lambda i: (i, 0)),
            pl.BlockSpec((H, H), lambda i: (0, 0)),
            pl.BlockSpec((H, H), lambda i: (0, 0)),
            pl.BlockSpec((1, H), lambda i: (0, 0)),
        ],
        out_specs=pl.BlockSpec((_R, H), lambda i: (i, 0)),
        out_shape=jax.ShapeDtypeStruct((N, H), jnp.float32),
    )(agg, cnt, h, wlT, wrT, b)


def kernel(x, edge_index, W1l, b1, W1r, W2l, b2, W2r, W3l, b3, W3r):
    src = edge_index[0]
    dst = edge_index[1]
    ar = jnp.arange(PAD, dtype=jnp.int32)
    srcp = jnp.concatenate([src, (ar * 31) % N])
    dstp = jnp.concatenate([dst, NPAD + (ar & (TRASH - 1))])

    xpad = jnp.concatenate(
        [x, jnp.ones((N, 1), jnp.float32), jnp.zeros((N, CH - DIN - 1), jnp.float32)],
        axis=1,
    )
    w1lT = jnp.pad(W1l, ((0, 0), (0, CH - DIN))).T
    w1rT = jnp.pad(W1r, ((0, 0), (0, CH - DIN))).T

    p = _sc_agg(xpad, srcp, dstp, hid=False)[:, :N]
    h, cnt = _l1_dense(xpad, p[0], p[1], w1lT, w1rT, b1.reshape(1, H))

    for Wl, b, Wr in ((W2l, b2, W2r), (W3l, b3, W3r)):
        agg4 = _sc_agg(h.reshape(NCH * N, CH), srcp, dstp, hid=True)[:, :N]
        agg = jnp.transpose(agg4, (1, 0, 2)).reshape(N, H)
        h = _hid_dense(agg, cnt, h, Wl.T, Wr.T, b.reshape(1, H))
    return h


# speedup vs baseline: 10.4097x; 1.3100x over previous
import functools

import jax
import jax.numpy as jnp
from jax import lax
from jax.experimental import pallas as pl
from jax.experimental.pallas import tpu as pltpu
from jax.experimental.pallas import tpu_sc as plsc

N = 50000
E = 800000
DIN = 26
H = 128
CH = 32
NCH = H // CH
NSUB = 16
NCORE = 2
W = 128
GSTEPS = 5
GE = GSTEPS * W
EP = 819200
PAD = EP - E
TRASH = 64
NPAD = 50176
SLAB_ROWS = NPAD + TRASH
STRIPE = NPAD // NSUB
ZR = 196


def _sc_agg(table, srcp, dstp, hid):
    mesh = plsc.VectorSubcoreMesh(core_axis_name="c", subcore_axis_name="s")
    if hid:
        nout = NCH
        ept = EP // NSUB
    else:
        nout = 2
        ept = EP // (NSUB * NCORE)
    ngroups = ept // GE

    @functools.partial(
        pl.kernel,
        mesh=mesh,
        compiler_params=pltpu.CompilerParams(use_tc_tiling_on_sc=False),
        out_type=(jax.ShapeDtypeStruct((NPAD, H), jnp.float32) if hid else
                  jax.ShapeDtypeStruct((nout, NPAD, CH), jnp.float32)),
        scratch_types=[
            pltpu.VMEM_SHARED((SLAB_ROWS, CH), jnp.float32),
            pltpu.VMEM((2, GE), jnp.int32),
            pltpu.VMEM((2, GSTEPS, W), jnp.int32),
            pltpu.VMEM((ZR, CH), jnp.float32),
            pltpu.VMEM((GE, CH), jnp.float32),
        ] + [pltpu.SemaphoreType.DMA] * (2 * GSTEPS + 4),
    )
    def k(tab_hbm, src_hbm, dst_hbm, out_hbm, slab, sidx, didx, zbuf, rows, *sems):
        gsem = sems[:GSTEPS]
        ssem = sems[GSTEPS:2 * GSTEPS]
        isem = sems[2 * GSTEPS:]
        cid = lax.axis_index("c")
        sid = lax.axis_index("s")

        @pl.loop(0, ZR)
        def _(r):
            zbuf[r, pl.ds(0, 16)] = jnp.zeros((16,), jnp.float32)
            zbuf[r, pl.ds(16, 16)] = jnp.zeros((16,), jnp.float32)

        for p in range(2 if hid else 1):
            if hid:
                chunk = cid * 2 + p
                oslot = chunk
                ebase = sid * ept
            else:
                chunk = None
                oslot = cid
                ebase = (sid * NCORE + cid) * ept
            rbase = ebase // W

            plsc.subcore_barrier()
            for z in range(STRIPE // ZR):
                pltpu.sync_copy(zbuf, slab.at[pl.ds(sid * STRIPE + z * ZR, ZR)])
            plsc.subcore_barrier()

            def fire_idx(g, par):
                pltpu.async_copy(
                    src_hbm.at[pl.ds(ebase + g * GE, GE)], sidx.at[par],
                    isem[2 * par])
                pltpu.async_copy(
                    dst_hbm.at[pl.ds(rbase + g * GSTEPS, GSTEPS)], didx.at[par],
                    isem[2 * par + 1])

            def wait_idx(par):
                pltpu.make_async_copy(
                    src_hbm.at[pl.ds(ebase, GE)], sidx.at[par],
                    isem[2 * par]).wait()
                pltpu.make_async_copy(
                    dst_hbm.at[pl.ds(rbase, GSTEPS)], didx.at[par],
                    isem[2 * par + 1]).wait()

            def wait_scatters():
                for b in range(GSTEPS):
                    pltpu.make_async_copy(
                        tab_hbm.at[pl.ds(0, W)],
                        rows.at[pl.ds(b * W, W)], ssem[b]).wait()

            def group_body(g, par, other):
                @pl.when(g > 0)
                def _():
                    wait_scatters()
                @pl.when(g + 1 < ngroups)
                def _():
                    fire_idx(g + 1, other)
                wait_idx(par)
                handles = []
                for b in range(GSTEPS):
                    if hid:
                        @pl.loop(0, W, step=16)
                        def _(i, _b=b, _p=par):
                            sl = (_p, pl.ds(_b * W + i, 16))
                            sidx[sl] = sidx[sl] * NCH + chunk
                    handles.append(pltpu.async_copy(
                        tab_hbm.at[sidx.at[par, pl.ds(b * W, W)]],
                        rows.at[pl.ds(b * W, W)], gsem[b]))
                for b in range(GSTEPS):
                    handles[b].wait()
                    pltpu.async_copy(
                        rows.at[pl.ds(b * W, W)],
                        slab.at[didx.at[par, b]], ssem[b], add=True)

            fire_idx(0, 0)

            @pl.loop(0, ngroups // 2)
            def _(i):
                group_body(2 * i, 0, 1)
                group_body(2 * i + 1, 1, 0)

            wait_scatters()
            plsc.subcore_barrier()
            if hid:
                pltpu.sync_copy(
                    slab.at[pl.ds(sid * STRIPE, STRIPE)],
                    out_hbm.at[pl.ds(sid * STRIPE, STRIPE),
                               pl.ds(oslot * CH, CH)],
                )
            else:
                pltpu.sync_copy(
                    slab.at[pl.ds(sid * STRIPE, STRIPE)],
                    out_hbm.at[oslot, pl.ds(sid * STRIPE, STRIPE)],
                )

    return k(table, srcp, dstp.reshape(EP // W, W))


_DOT = dict(
    dimension_numbers=(((1,), (0,)), ((), ())),
    preferred_element_type=jnp.float32,
    precision=jax.lax.Precision.HIGHEST,
)

_R = 2000


def _l1_body(xp, p0, p1, wl, wr, bb, h_out, cnt_out):
    p = p0[...] + p1[...]
    cnt = p[:, DIN:DIN + 1]
    mean = p * (1.0 / jnp.maximum(cnt, 1.0))
    acc = lax.dot_general(mean, wl[...], **_DOT)
    acc = acc + lax.dot_general(xp[...], wr[...], **_DOT)
    h_out[...] = jnp.maximum(acc + bb[...], 0.0)
    cnt_out[...] = cnt


def _l1_dense(xpad, p0, p1, wlT, wrT, b):
    return pl.pallas_call(
        _l1_body,
        grid=(N // _R,),
        in_specs=[
            pl.BlockSpec((_R, CH), lambda i: (i, 0)),
            pl.BlockSpec((_R, CH), lambda i: (i, 0)),
            pl.BlockSpec((_R, CH), lambda i: (i, 0)),
            pl.BlockSpec((CH, H), lambda i: (0, 0)),
            pl.BlockSpec((CH, H), lambda i: (0, 0)),
            pl.BlockSpec((1, H), lambda i: (0, 0)),
        ],
        out_specs=[
            pl.BlockSpec((_R, H), lambda i: (i, 0)),
            pl.BlockSpec((_R, 1), lambda i: (i, 0)),
        ],
        out_shape=[
            jax.ShapeDtypeStruct((N, H), jnp.float32),
            jax.ShapeDtypeStruct((N, 1), jnp.float32),
        ],
    )(xpad, p0, p1, wlT, wrT, b)


def _hid_body(agg, cnt, h, wl, wr, bb, h_out):
    mean = agg[...] * (1.0 / jnp.maximum(cnt[...], 1.0))
    acc = lax.dot_general(mean, wl[...], **_DOT)
    acc = acc + lax.dot_general(h[...], wr[...], **_DOT)
    h_out[...] = jnp.maximum(acc + bb[...], 0.0)


def _hid_dense(agg, cnt, h, wlT, wrT, b):
    return pl.pallas_call(
        _hid_body,
        grid=(N // _R,),
        in_specs=[
            pl.BlockSpec((_R, H), lambda i: (i, 0)),
            pl.BlockSpec((_R, 1), lambda i: (i, 0)),
            pl.BlockSpec((_R, H), lambda i: (i, 0)),
            pl.BlockSpec((H, H), lambda i: (0, 0)),
            pl.BlockSpec((H, H), lambda i: (0, 0)),
            pl.BlockSpec((1, H), lambda i: (0, 0)),
        ],
        out_specs=pl.BlockSpec((_R, H), lambda i: (i, 0)),
        out_shape=jax.ShapeDtypeStruct((N, H), jnp.float32),
    )(agg, cnt, h, wlT, wrT, b)


def kernel(x, edge_index, W1l, b1, W1r, W2l, b2, W2r, W3l, b3, W3r):
    src = edge_index[0]
    dst = edge_index[1]
    ar = jnp.arange(PAD, dtype=jnp.int32)
    srcp = jnp.concatenate([src, (ar * 31) % N])
    dstp = jnp.concatenate([dst, NPAD + (ar & (TRASH - 1))])

    xpad = jnp.concatenate(
        [x, jnp.ones((N, 1), jnp.float32), jnp.zeros((N, CH - DIN - 1), jnp.float32)],
        axis=1,
    )
    w1lT = jnp.pad(W1l, ((0, 0), (0, CH - DIN))).T
    w1rT = jnp.pad(W1r, ((0, 0), (0, CH - DIN))).T

    p = _sc_agg(xpad, srcp, dstp, hid=False)[:, :N]
    h, cnt = _l1_dense(xpad, p[0], p[1], w1lT, w1rT, b1.reshape(1, H))

    for Wl, b, Wr in ((W2l, b2, W2r), (W3l, b3, W3r)):
        agg = _sc_agg(h.reshape(NCH * N, CH), srcp, dstp, hid=True)[:N]
        h = _hid_dense(agg, cnt, h, Wl.T, Wr.T, b.reshape(1, H))
    return h


# speedup vs baseline: 10.6420x; 1.0223x over previous
import functools

import jax
import jax.numpy as jnp
from jax import lax
from jax.experimental import pallas as pl
from jax.experimental.pallas import tpu as pltpu
from jax.experimental.pallas import tpu_sc as plsc

N = 50000
E = 800000
DIN = 26
H = 128
CH = 32
NCH = H // CH
NSUB = 16
NCORE = 2
W = 128
GSTEPS = 5
GE = GSTEPS * W
EP = 819200
PAD = EP - E
TRASH = 64
NPAD = 50176
SLAB_ROWS = NPAD + TRASH
STRIPE = NPAD // NSUB
ZR = 196


def _sc_agg(table, srcp, dstp, hid):
    mesh = plsc.VectorSubcoreMesh(core_axis_name="c", subcore_axis_name="s")
    if hid:
        nout = NCH
        ept = EP // NSUB
    else:
        nout = 2
        ept = EP // (NSUB * NCORE)
    ngroups = ept // GE

    @functools.partial(
        pl.kernel,
        mesh=mesh,
        compiler_params=pltpu.CompilerParams(use_tc_tiling_on_sc=False),
        out_type=(jax.ShapeDtypeStruct((NPAD, H), jnp.float32) if hid else
                  jax.ShapeDtypeStruct((nout, NPAD, CH), jnp.float32)),
        scratch_types=[
            pltpu.VMEM_SHARED((SLAB_ROWS, CH), jnp.float32),
            pltpu.VMEM((2, GE), jnp.int32),
            pltpu.VMEM((2, GSTEPS, W), jnp.int32),
            pltpu.VMEM((ZR, CH), jnp.float32),
            pltpu.VMEM((GE, CH), jnp.float32),
        ] + [pltpu.SemaphoreType.DMA] * (2 * GSTEPS + 4),
    )
    def k(tab_hbm, src_hbm, dst_hbm, out_hbm, slab, sidx, didx, zbuf, rows, *sems):
        gsem = sems[:GSTEPS]
        ssem = sems[GSTEPS:2 * GSTEPS]
        isem = sems[2 * GSTEPS:]
        cid = lax.axis_index("c")
        sid = lax.axis_index("s")

        @pl.loop(0, ZR)
        def _(r):
            zbuf[r, pl.ds(0, 16)] = jnp.zeros((16,), jnp.float32)
            zbuf[r, pl.ds(16, 16)] = jnp.zeros((16,), jnp.float32)

        for p in range(2 if hid else 1):
            if hid:
                chunk = cid * 2 + p
                oslot = chunk
                ebase = sid * ept
            else:
                chunk = None
                oslot = cid
                ebase = (sid * NCORE + cid) * ept
            rbase = ebase // W

            plsc.subcore_barrier()
            for z in range(STRIPE // ZR):
                pltpu.sync_copy(zbuf, slab.at[pl.ds(sid * STRIPE + z * ZR, ZR)])
            plsc.subcore_barrier()

            def fire_idx(g, par):
                pltpu.async_copy(
                    src_hbm.at[pl.ds(ebase + g * GE, GE)], sidx.at[par],
                    isem[2 * par])
                pltpu.async_copy(
                    dst_hbm.at[pl.ds(rbase + g * GSTEPS, GSTEPS)], didx.at[par],
                    isem[2 * par + 1])

            def wait_idx(par):
                pltpu.make_async_copy(
                    src_hbm.at[pl.ds(ebase, GE)], sidx.at[par],
                    isem[2 * par]).wait()
                pltpu.make_async_copy(
                    dst_hbm.at[pl.ds(rbase, GSTEPS)], didx.at[par],
                    isem[2 * par + 1]).wait()

            def wait_scatters():
                for b in range(GSTEPS):
                    pltpu.make_async_copy(
                        tab_hbm.at[pl.ds(0, W)],
                        rows.at[pl.ds(b * W, W)], ssem[b]).wait()

            def group_body(g, par, other):
                @pl.when(g > 0)
                def _():
                    wait_scatters()
                @pl.when(g + 1 < ngroups)
                def _():
                    fire_idx(g + 1, other)
                wait_idx(par)
                handles = []
                for b in range(GSTEPS):
                    if hid:
                        @pl.loop(0, W, step=16)
                        def _(i, _b=b, _p=par):
                            sl = (_p, pl.ds(_b * W + i, 16))
                            sidx[sl] = sidx[sl] * NCH + chunk
                    handles.append(pltpu.async_copy(
                        tab_hbm.at[sidx.at[par, pl.ds(b * W, W)]],
                        rows.at[pl.ds(b * W, W)], gsem[b]))
                for b in range(GSTEPS):
                    handles[b].wait()
                    pltpu.async_copy(
                        rows.at[pl.ds(b * W, W)],
                        slab.at[didx.at[par, b]], ssem[b], add=True)

            fire_idx(0, 0)

            @pl.loop(0, ngroups // 2)
            def _(i):
                group_body(2 * i, 0, 1)
                group_body(2 * i + 1, 1, 0)

            wait_scatters()
            plsc.subcore_barrier()
            if hid:
                pltpu.sync_copy(
                    slab.at[pl.ds(sid * STRIPE, STRIPE)],
                    out_hbm.at[pl.ds(sid * STRIPE, STRIPE),
                               pl.ds(oslot * CH, CH)],
                )
            else:
                pltpu.sync_copy(
                    slab.at[pl.ds(sid * STRIPE, STRIPE)],
                    out_hbm.at[oslot, pl.ds(sid * STRIPE, STRIPE)],
                )

    return k(table, srcp, dstp.reshape(EP // W, W))


_DOT = dict(
    dimension_numbers=(((1,), (0,)), ((), ())),
    preferred_element_type=jnp.float32,
    precision=jax.lax.Precision.HIGHEST,
)

_R = 2000


def _root_body(h, wr, bb, o):
    o[...] = lax.dot_general(h[...], wr[...], **_DOT) + bb[...]


def _root(h, wrT, b):
    kin = h.shape[1]
    return pl.pallas_call(
        _root_body,
        grid=(N // _R,),
        in_specs=[
            pl.BlockSpec((_R, kin), lambda i: (i, 0)),
            pl.BlockSpec((kin, H), lambda i: (0, 0)),
            pl.BlockSpec((1, H), lambda i: (0, 0)),
        ],
        out_specs=pl.BlockSpec((_R, H), lambda i: (i, 0)),
        out_shape=jax.ShapeDtypeStruct((N, H), jnp.float32),
    )(h, wrT, b)


def _l1_body(p0, p1, wl, root, h_out, cnt_out):
    p = p0[...] + p1[...]
    cnt = p[:, DIN:DIN + 1]
    mean = p * (1.0 / jnp.maximum(cnt, 1.0))
    acc = lax.dot_general(mean, wl[...], **_DOT)
    h_out[...] = jnp.maximum(acc + root[...], 0.0)
    cnt_out[...] = cnt


def _l1_combine(p0, p1, wlT, root):
    return pl.pallas_call(
        _l1_body,
        grid=(N // _R,),
        in_specs=[
            pl.BlockSpec((_R, CH), lambda i: (i, 0)),
            pl.BlockSpec((_R, CH), lambda i: (i, 0)),
            pl.BlockSpec((CH, H), lambda i: (0, 0)),
            pl.BlockSpec((_R, H), lambda i: (i, 0)),
        ],
        out_specs=[
            pl.BlockSpec((_R, H), lambda i: (i, 0)),
            pl.BlockSpec((_R, 1), lambda i: (i, 0)),
        ],
        out_shape=[
            jax.ShapeDtypeStruct((N, H), jnp.float32),
            jax.ShapeDtypeStruct((N, 1), jnp.float32),
        ],
    )(p0, p1, wlT, root)


def _hid_body(agg, cnt, wl, root, h_out):
    mean = agg[...] * (1.0 / jnp.maximum(cnt[...], 1.0))
    acc = lax.dot_general(mean, wl[...], **_DOT)
    h_out[...] = jnp.maximum(acc + root[...], 0.0)


def _hid_combine(agg, cnt, wlT, root):
    return pl.pallas_call(
        _hid_body,
        grid=(N // _R,),
        in_specs=[
            pl.BlockSpec((_R, H), lambda i: (i, 0)),
            pl.BlockSpec((_R, 1), lambda i: (i, 0)),
            pl.BlockSpec((H, H), lambda i: (0, 0)),
            pl.BlockSpec((_R, H), lambda i: (i, 0)),
        ],
        out_specs=pl.BlockSpec((_R, H), lambda i: (i, 0)),
        out_shape=jax.ShapeDtypeStruct((N, H), jnp.float32),
    )(agg, cnt, wlT, root)


def kernel(x, edge_index, W1l, b1, W1r, W2l, b2, W2r, W3l, b3, W3r):
    src = edge_index[0]
    dst = edge_index[1]
    ar = jnp.arange(PAD, dtype=jnp.int32)
    srcp = jnp.concatenate([src, (ar * 31) % N])
    dstp = jnp.concatenate([dst, NPAD + (ar & (TRASH - 1))])

    xpad = jnp.concatenate(
        [x, jnp.ones((N, 1), jnp.float32), jnp.zeros((N, CH - DIN - 1), jnp.float32)],
        axis=1,
    )
    w1lT = jnp.pad(W1l, ((0, 0), (0, CH - DIN))).T
    w1rT = jnp.pad(W1r, ((0, 0), (0, CH - DIN))).T

    p = _sc_agg(xpad, srcp, dstp, hid=False)[:, :N]
    root = _root(xpad, w1rT, b1.reshape(1, H))
    h, cnt = _l1_combine(p[0], p[1], w1lT, root)

    for Wl, b, Wr in ((W2l, b2, W2r), (W3l, b3, W3r)):
        agg = _sc_agg(h.reshape(NCH * N, CH), srcp, dstp, hid=True)[:N]
        root = _root(h, Wr.T, b.reshape(1, H))
        h = _hid_combine(agg, cnt, Wl.T, root)
    return h


# speedup vs baseline: 10.8255x; 1.0172x over previous
import functools

import jax
import jax.numpy as jnp
from jax import lax
from jax.experimental import pallas as pl
from jax.experimental.pallas import tpu as pltpu
from jax.experimental.pallas import tpu_sc as plsc

N = 50000
E = 800000
DIN = 26
H = 128
CH = 32
NCH = H // CH
NSUB = 16
NCORE = 2
W = 128
GSTEPS = 5
GE = GSTEPS * W
EP = 819200
PAD = EP - E
TRASH = 64
NPAD = 50176
SLAB_ROWS = NPAD + TRASH
STRIPE = NPAD // NSUB
ZR = 196


def _sc_agg(table, srcp, dstp, hid):
    mesh = plsc.VectorSubcoreMesh(core_axis_name="c", subcore_axis_name="s")
    if hid:
        nout = NCH
        ept = EP // NSUB
    else:
        nout = 2
        ept = EP // (NSUB * NCORE)
    ngroups = ept // GE

    @functools.partial(
        pl.kernel,
        mesh=mesh,
        compiler_params=pltpu.CompilerParams(use_tc_tiling_on_sc=False),
        out_type=(jax.ShapeDtypeStruct((NPAD, H), jnp.float32) if hid else
                  jax.ShapeDtypeStruct((nout, NPAD, CH), jnp.float32)),
        scratch_types=[
            pltpu.VMEM_SHARED((SLAB_ROWS, CH), jnp.float32),
            pltpu.VMEM((2, GE), jnp.int32),
            pltpu.VMEM((2, GSTEPS, W), jnp.int32),
            pltpu.VMEM((ZR, CH), jnp.float32),
            pltpu.VMEM((GE, CH), jnp.float32),
        ] + [pltpu.SemaphoreType.DMA] * (2 * GSTEPS + 4),
    )
    def k(tab_hbm, src_hbm, dst_hbm, out_hbm, slab, sidx, didx, zbuf, rows, *sems):
        gsem = sems[:GSTEPS]
        ssem = sems[GSTEPS:2 * GSTEPS]
        isem = sems[2 * GSTEPS:]
        cid = lax.axis_index("c")
        sid = lax.axis_index("s")

        @pl.loop(0, ZR)
        def _(r):
            zbuf[r, pl.ds(0, 16)] = jnp.zeros((16,), jnp.float32)
            zbuf[r, pl.ds(16, 16)] = jnp.zeros((16,), jnp.float32)

        for p in range(2 if hid else 1):
            if hid:
                chunk = cid * 2 + p
                oslot = chunk
                ebase = sid * ept
            else:
                chunk = None
                oslot = cid
                ebase = (sid * NCORE + cid) * ept
            rbase = ebase // W

            plsc.subcore_barrier()
            for z in range(STRIPE // ZR):
                pltpu.sync_copy(zbuf, slab.at[pl.ds(sid * STRIPE + z * ZR, ZR)])
            plsc.subcore_barrier()

            def fire_idx(g, par):
                pltpu.async_copy(
                    src_hbm.at[pl.ds(ebase + g * GE, GE)], sidx.at[par],
                    isem[2 * par])
                pltpu.async_copy(
                    dst_hbm.at[pl.ds(rbase + g * GSTEPS, GSTEPS)], didx.at[par],
                    isem[2 * par + 1])

            def wait_idx(par):
                pltpu.make_async_copy(
                    src_hbm.at[pl.ds(ebase, GE)], sidx.at[par],
                    isem[2 * par]).wait()
                pltpu.make_async_copy(
                    dst_hbm.at[pl.ds(rbase, GSTEPS)], didx.at[par],
                    isem[2 * par + 1]).wait()

            def wait_scatters():
                for b in range(GSTEPS):
                    pltpu.make_async_copy(
                        tab_hbm.at[pl.ds(0, W)],
                        rows.at[pl.ds(b * W, W)], ssem[b]).wait()

            def group_body(g, par, other):
                @pl.when(g > 0)
                def _():
                    wait_scatters()
                @pl.when(g + 1 < ngroups)
                def _():
                    fire_idx(g + 1, other)
                wait_idx(par)
                handles = []
                for b in range(GSTEPS):
                    if hid:
                        @pl.loop(0, W, step=16)
                        def _(i, _b=b, _p=par):
                            sl = (_p, pl.ds(_b * W + i, 16))
                            sidx[sl] = sidx[sl] * NCH + chunk
                    handles.append(pltpu.async_copy(
                        tab_hbm.at[sidx.at[par, pl.ds(b * W, W)]],
                        rows.at[pl.ds(b * W, W)], gsem[b]))
                for b in range(GSTEPS):
                    handles[b].wait()
                    pltpu.async_copy(
                        rows.at[pl.ds(b * W, W)],
                        slab.at[didx.at[par, b]], ssem[b], add=True)

            fire_idx(0, 0)

            @pl.loop(0, ngroups // 2)
            def _(i):
                group_body(2 * i, 0, 1)
                group_body(2 * i + 1, 1, 0)

            wait_scatters()
            plsc.subcore_barrier()
            if hid:
                pltpu.sync_copy(
                    slab.at[pl.ds(sid * STRIPE, STRIPE)],
                    out_hbm.at[pl.ds(sid * STRIPE, STRIPE),
                               pl.ds(oslot * CH, CH)],
                )
            else:
                pltpu.sync_copy(
                    slab.at[pl.ds(sid * STRIPE, STRIPE)],
                    out_hbm.at[oslot, pl.ds(sid * STRIPE, STRIPE)],
                )

    return k(table, srcp, dstp.reshape(EP // W, W))


_DOT = dict(
    dimension_numbers=(((1,), (0,)), ((), ())),
    preferred_element_type=jnp.float32,
    precision=jax.lax.Precision.DEFAULT,
)

_R = 2000


def _root_body(h, wr, bb, o):
    o[...] = lax.dot_general(h[...], wr[...], **_DOT) + bb[...]


def _root(h, wrT, b):
    kin = h.shape[1]
    return pl.pallas_call(
        _root_body,
        grid=(N // _R,),
        in_specs=[
            pl.BlockSpec((_R, kin), lambda i: (i, 0)),
            pl.BlockSpec((kin, H), lambda i: (0, 0)),
            pl.BlockSpec((1, H), lambda i: (0, 0)),
        ],
        out_specs=pl.BlockSpec((_R, H), lambda i: (i, 0)),
        out_shape=jax.ShapeDtypeStruct((N, H), jnp.float32),
    )(h, wrT, b)


def _l1_body(p0, p1, wl, root, h_out, cnt_out):
    p = p0[...] + p1[...]
    cnt = p[:, DIN:DIN + 1]
    mean = p * (1.0 / jnp.maximum(cnt, 1.0))
    acc = lax.dot_general(mean, wl[...], **_DOT)
    h_out[...] = jnp.maximum(acc + root[...], 0.0)
    cnt_out[...] = cnt


def _l1_combine(p0, p1, wlT, root):
    return pl.pallas_call(
        _l1_body,
        grid=(N // _R,),
        in_specs=[
            pl.BlockSpec((_R, CH), lambda i: (i, 0)),
            pl.BlockSpec((_R, CH), lambda i: (i, 0)),
            pl.BlockSpec((CH, H), lambda i: (0, 0)),
            pl.BlockSpec((_R, H), lambda i: (i, 0)),
        ],
        out_specs=[
            pl.BlockSpec((_R, H), lambda i: (i, 0)),
            pl.BlockSpec((_R, 1), lambda i: (i, 0)),
        ],
        out_shape=[
            jax.ShapeDtypeStruct((N, H), jnp.float32),
            jax.ShapeDtypeStruct((N, 1), jnp.float32),
        ],
    )(p0, p1, wlT, root)


def _hid_body(agg, cnt, wl, root, h_out):
    mean = agg[...] * (1.0 / jnp.maximum(cnt[...], 1.0))
    acc = lax.dot_general(mean, wl[...], **_DOT)
    h_out[...] = jnp.maximum(acc + root[...], 0.0)


def _hid_combine(agg, cnt, wlT, root):
    return pl.pallas_call(
        _hid_body,
        grid=(N // _R,),
        in_specs=[
            pl.BlockSpec((_R, H), lambda i: (i, 0)),
            pl.BlockSpec((_R, 1), lambda i: (i, 0)),
            pl.BlockSpec((H, H), lambda i: (0, 0)),
            pl.BlockSpec((_R, H), lambda i: (i, 0)),
        ],
        out_specs=pl.BlockSpec((_R, H), lambda i: (i, 0)),
        out_shape=jax.ShapeDtypeStruct((N, H), jnp.float32),
    )(agg, cnt, wlT, root)


def kernel(x, edge_index, W1l, b1, W1r, W2l, b2, W2r, W3l, b3, W3r):
    src = edge_index[0]
    dst = edge_index[1]
    ar = jnp.arange(PAD, dtype=jnp.int32)
    srcp = jnp.concatenate([src, (ar * 31) % N])
    dstp = jnp.concatenate([dst, NPAD + (ar & (TRASH - 1))])

    xpad = jnp.concatenate(
        [x, jnp.ones((N, 1), jnp.float32), jnp.zeros((N, CH - DIN - 1), jnp.float32)],
        axis=1,
    )
    w1lT = jnp.pad(W1l, ((0, 0), (0, CH - DIN))).T
    w1rT = jnp.pad(W1r, ((0, 0), (0, CH - DIN))).T

    p = _sc_agg(xpad, srcp, dstp, hid=False)[:, :N]
    root = _root(xpad, w1rT, b1.reshape(1, H))
    h, cnt = _l1_combine(p[0], p[1], w1lT, root)

    for Wl, b, Wr in ((W2l, b2, W2r), (W3l, b3, W3r)):
        agg = _sc_agg(h.reshape(NCH * N, CH), srcp, dstp, hid=True)[:N]
        root = _root(h, Wr.T, b.reshape(1, H))
        h = _hid_combine(agg, cnt, Wl.T, root)
    return h
